# baseline (speedup 1.0000x reference)
import functools

import jax
import jax.numpy as jnp
from jax import lax
from jax.experimental import pallas as pl
from jax.experimental.pallas import tpu as pltpu
from jax.experimental.pallas import tpu_sc as plsc

_N = 10000
_E = 320000
_D = 128
_DE = 16
_NB = 64
_EPS = 1e-5
_NC = 2
_NS = 16
_NW = _NC * _NS
_CHUNK = 128
_CPW = 81
_EPW = _CPW * _CHUNK
_EPAD = _NW * _EPW
_CPWE = 80
_EPADE = _NW * _CPWE * _CHUNK
_HALFE = _CPWE // 2
_NSH = 10112
_RPT = _NSH // _NS
_ZCH = _NSH // _CHUNK
_TW = 32
_R = 1000
_G = _N // _R

def _mesh():
    return plsc.VectorSubcoreMesh(
        core_axis_name="c", subcore_axis_name="s",
        num_cores=_NC, num_subcores=_NS)


def _zero_buf(buf, width):
    def zrow(i, carry):
        for j in range(width // 16):
            buf[i, pl.ds(j * 16, 16)] = jnp.zeros((16,), jnp.float32)
        return carry
    lax.fori_loop(0, _CHUNK, zrow, 0)


def _zero_shared(s, zbuf, sh):
    def zch(t, carry):
        ch = s + t * _NS

        @pl.when(ch < _ZCH)
        def _():
            pltpu.sync_copy(zbuf, sh.at[pl.ds(ch * _CHUNK, _CHUNK)])
        return carry
    lax.fori_loop(0, (_ZCH + _NS - 1) // _NS, zch, 0)


@functools.lru_cache(maxsize=None)
def _make_row_scatter():
    scratch = [
        pltpu.VMEM((3, _CHUNK), jnp.int32),
        pltpu.VMEM((3, _CHUNK), jnp.int32),
        pltpu.VMEM((_CHUNK, _D), jnp.float32),
        pltpu.VMEM((_CHUNK, _D), jnp.float32),
        pltpu.VMEM((_CHUNK, _D), jnp.float32),
        pltpu.VMEM_SHARED((_NSH, _D), jnp.float32),
        pltpu.SemaphoreType.DMA,
        pltpu.SemaphoreType.DMA,
        pltpu.SemaphoreType.DMA,
        pltpu.SemaphoreType.DMA,
        pltpu.SemaphoreType.DMA,
        pltpu.SemaphoreType.DMA,
    ]

    def body(x_hbm, src_hbm, dst_hbm, s_out,
             src_v, dst_v, rows0, rows1, rows2, s_sh,
             gs0, gs1, gs2, ss0, ss1, ss2):
        c = lax.axis_index("c")
        s = lax.axis_index("s")
        wid = s * _NC + c
        bufs = (rows0, rows1, rows2)
        gsem = (gs0, gs1, gs2)
        ssem = (ss0, ss1, ss2)

        _zero_buf(rows0, _D)
        _zero_shared(s, rows0, s_sh)
        plsc.subcore_barrier()

        pltpu.sync_copy(src_hbm.at[wid, 0], src_v.at[0])
        pltpu.sync_copy(dst_hbm.at[wid, 0], dst_v.at[0])
        pltpu.sync_copy(src_hbm.at[wid, 1], src_v.at[1])
        pltpu.sync_copy(dst_hbm.at[wid, 1], dst_v.at[1])
        pltpu.async_copy(x_hbm.at[src_v.at[0]], rows0, gs0)
        pltpu.async_copy(x_hbm.at[src_v.at[1]], rows1, gs1)

        def step(t, carry):
            for b in range(3):
                j = 3 * t + b
                b2 = (b + 2) % 3
                pltpu.make_async_copy(
                    x_hbm.at[src_v.at[b]], bufs[b], gsem[b]).wait()
                pltpu.sync_copy(bufs[b], s_sh.at[dst_v.at[b]], add=True)

                def prefetch():
                    pltpu.sync_copy(src_hbm.at[wid, j + 2], src_v.at[b2])
                    pltpu.sync_copy(dst_hbm.at[wid, j + 2], dst_v.at[b2])
                    pltpu.async_copy(x_hbm.at[src_v.at[b2]], bufs[b2], gsem[b2])
                if b == 0:
                    prefetch()
                else:
                    pl.when(t + 1 < _CPW // 3)(prefetch)
            return carry
        lax.fori_loop(0, _CPW // 3, step, 0)
        plsc.subcore_barrier()

        pltpu.sync_copy(s_sh.at[pl.ds(s * _RPT, _RPT)],
                        s_out.at[c, pl.ds(s * _RPT, _RPT)])

    return pl.kernel(
        body, mesh=_mesh(),
        out_type=jax.ShapeDtypeStruct((_NC, _NSH, _D), jnp.float32),
        scratch_types=scratch)


@functools.lru_cache(maxsize=None)
def _make_ea_scatter():
    scratch = [
        pltpu.VMEM((_CPWE, _CHUNK), jnp.int32),
        pltpu.VMEM((_CHUNK, _TW), jnp.float32),
        pltpu.VMEM((_CHUNK, _TW), jnp.float32),
        pltpu.VMEM_SHARED((_NSH, _TW), jnp.float32),
        pltpu.SemaphoreType.DMA,
        pltpu.SemaphoreType.DMA,
    ]

    def body(ea_hbm, dst_hbm, t_out, dst_v, ea0, ea1, t_sh, sem0, sem1):
        c = lax.axis_index("c")
        s = lax.axis_index("s")
        wid = s * _NC + c

        _zero_buf(ea0, _TW)
        pltpu.sync_copy(dst_hbm.at[wid], dst_v)
        _zero_shared(s, ea0, t_sh)
        plsc.subcore_barrier()

        pltpu.async_copy(ea_hbm.at[wid, 0], ea0, sem0)

        def step(t, carry):
            j0 = 2 * t
            j1 = j0 + 1
            pltpu.async_copy(ea_hbm.at[wid, j1], ea1, sem1)
            pltpu.make_async_copy(ea_hbm.at[wid, j0], ea0, sem0).wait()
            pltpu.sync_copy(ea0, t_sh.at[dst_v.at[j0]], add=True)

            @pl.when(t + 1 < _HALFE)
            def _():
                pltpu.async_copy(ea_hbm.at[wid, j0 + 2], ea0, sem0)
            pltpu.make_async_copy(ea_hbm.at[wid, j1], ea1, sem1).wait()
            pltpu.sync_copy(ea1, t_sh.at[dst_v.at[j1]], add=True)
            return carry
        lax.fori_loop(0, _HALFE, step, 0)
        plsc.subcore_barrier()

        pltpu.sync_copy(t_sh.at[pl.ds(s * _RPT, _RPT)],
                        t_out.at[c, pl.ds(s * _RPT, _RPT)])

    return pl.kernel(
        body, mesh=_mesh(),
        out_type=jax.ShapeDtypeStruct((_NC, _NSH, _TW), jnp.float32),
        scratch_types=scratch)


def _dense_body(x_ref, s_ref, t_ref, wd_ref, ws_ref, we_ref, b_ref,
                hpre_ref, stats_ref, sum_ref, sq_ref):
    i = pl.program_id(0)
    x = x_ref[...]
    ssum = s_ref[0] + s_ref[1] + x
    tb = t_ref[0] + t_ref[1]
    deg = tb[:, 16:17] + 1.0
    eap = tb[:, :16] + 1.0
    agg = deg * (jnp.dot(x, wd_ref[...], preferred_element_type=jnp.float32,
                         precision=lax.Precision.HIGHEST) + b_ref[...])
    agg = agg + jnp.dot(ssum, ws_ref[...], preferred_element_type=jnp.float32,
                        precision=lax.Precision.HIGHEST)
    agg = agg + jnp.dot(eap, we_ref[...], preferred_element_type=jnp.float32,
                        precision=lax.Precision.HIGHEST)
    hp = jnp.maximum(agg, 0.0)
    hpre_ref[...] = hp

    @pl.when(i == 0)
    def _():
        sum_ref[...] = jnp.zeros_like(sum_ref)
        sq_ref[...] = jnp.zeros_like(sq_ref)
    sum_ref[...] += jnp.sum(hp, axis=0, keepdims=True)
    sq_ref[...] += jnp.sum(hp * hp, axis=0, keepdims=True)

    @pl.when(i == _G - 1)
    def _():
        stats_ref[...] = jnp.concatenate([sum_ref[...], sq_ref[...]], axis=0)


def _layer_dense(x, s_pair, t_pair, wdt, wst, wet, b):
    return pl.pallas_call(
        _dense_body,
        grid=(_G,),
        in_specs=[
            pl.BlockSpec((_R, _D), lambda i: (i, 0)),
            pl.BlockSpec((2, _R, _D), lambda i: (0, i, 0)),
            pl.BlockSpec((2, _R, _TW), lambda i: (0, i, 0)),
            pl.BlockSpec((_D, _D), lambda i: (0, 0)),
            pl.BlockSpec((_D, _D), lambda i: (0, 0)),
            pl.BlockSpec((_DE, _D), lambda i: (0, 0)),
            pl.BlockSpec((1, _D), lambda i: (0, 0)),
        ],
        out_specs=[
            pl.BlockSpec((_R, _D), lambda i: (i, 0)),
            pl.BlockSpec((2, _D), lambda i: (0, 0)),
        ],
        out_shape=[
            jax.ShapeDtypeStruct((_N, _D), jnp.float32),
            jax.ShapeDtypeStruct((2, _D), jnp.float32),
        ],
        scratch_shapes=[
            pltpu.VMEM((1, _D), jnp.float32),
            pltpu.VMEM((1, _D), jnp.float32),
        ],
    )(x, s_pair, t_pair, wdt, wst, wet, b)


def _bn_scale_shift(stats, g, be):
    mu = stats[0:1] / _N
    var = stats[1:2] / _N - mu * mu
    inv = g * lax.rsqrt(var + _EPS)
    return inv, be - mu * inv


def _bn_body(hp_ref, st_ref, g_ref, be_ref, h_ref):
    inv, shift = _bn_scale_shift(st_ref[...], g_ref[...], be_ref[...])
    h_ref[...] = jnp.maximum(hp_ref[...] * inv + shift, 0.0)


def _bn_apply(hpre, stats, g, be):
    return pl.pallas_call(
        _bn_body,
        grid=(_G,),
        in_specs=[
            pl.BlockSpec((_R, _D), lambda i: (i, 0)),
            pl.BlockSpec((2, _D), lambda i: (0, 0)),
            pl.BlockSpec((1, _D), lambda i: (0, 0)),
            pl.BlockSpec((1, _D), lambda i: (0, 0)),
        ],
        out_specs=pl.BlockSpec((_R, _D), lambda i: (i, 0)),
        out_shape=jax.ShapeDtypeStruct((_N, _D), jnp.float32),
    )(hpre, stats, g, be)


def _onehot(batch_block):
    io = lax.broadcasted_iota(jnp.int32, (_R, _NB), 1)
    return (batch_block == io).astype(jnp.float32)


def _bn_pool_body(hp_ref, st_ref, g_ref, be_ref, bt_ref, h_ref, m_ref, macc):
    i = pl.program_id(0)
    inv, shift = _bn_scale_shift(st_ref[...], g_ref[...], be_ref[...])
    h = jnp.maximum(hp_ref[...] * inv + shift, 0.0)
    h_ref[...] = h
    oh = _onehot(bt_ref[...])
    contrib = lax.dot_general(oh, h, (((0,), (0,)), ((), ())),
                              preferred_element_type=jnp.float32,
                              precision=lax.Precision.HIGHEST)

    @pl.when(i == 0)
    def _():
        macc[...] = jnp.zeros_like(macc)
    macc[...] += contrib

    @pl.when(i == _G - 1)
    def _():
        m_ref[...] = macc[...]


def _bn_pool(hpre, stats, g, be, batch2d):
    return pl.pallas_call(
        _bn_pool_body,
        grid=(_G,),
        in_specs=[
            pl.BlockSpec((_R, _D), lambda i: (i, 0)),
            pl.BlockSpec((2, _D), lambda i: (0, 0)),
            pl.BlockSpec((1, _D), lambda i: (0, 0)),
            pl.BlockSpec((1, _D), lambda i: (0, 0)),
            pl.BlockSpec((_R, 1), lambda i: (i, 0)),
        ],
        out_specs=[
            pl.BlockSpec((_R, _D), lambda i: (i, 0)),
            pl.BlockSpec((_NB, _D), lambda i: (0, 0)),
        ],
        out_shape=[
            jax.ShapeDtypeStruct((_N, _D), jnp.float32),
            jax.ShapeDtypeStruct((_NB, _D), jnp.float32),
        ],
        scratch_shapes=[pltpu.VMEM((_NB, _D), jnp.float32)],
    )(hpre, stats, g, be, batch2d)


def _final_body(h_ref, bt_ref, m_ref, fa_ref, fb_ref, fb1_ref, fw2_ref,
                fb2_ref, out_ref):
    oh = _onehot(bt_ref[...])
    rep = jnp.dot(oh, m_ref[...], preferred_element_type=jnp.float32,
                  precision=lax.Precision.HIGHEST)
    z = jnp.dot(h_ref[...], fa_ref[...], preferred_element_type=jnp.float32,
                precision=lax.Precision.HIGHEST)
    z = z + jnp.dot(rep, fb_ref[...], preferred_element_type=jnp.float32,
                    precision=lax.Precision.HIGHEST)
    z = jnp.maximum(z + fb1_ref[...], 0.0)
    out_ref[...] = jnp.dot(z, fw2_ref[...], preferred_element_type=jnp.float32,
                           precision=lax.Precision.HIGHEST) + fb2_ref[...]


def _final(h2, batch2d, m, fat, fbt, fb1, fw2t, fb2):
    mlp = fat.shape[1]
    c = fw2t.shape[1]
    return pl.pallas_call(
        _final_body,
        grid=(_G,),
        in_specs=[
            pl.BlockSpec((_R, _D), lambda i: (i, 0)),
            pl.BlockSpec((_R, 1), lambda i: (i, 0)),
            pl.BlockSpec((_NB, _D), lambda i: (0, 0)),
            pl.BlockSpec((_D, mlp), lambda i: (0, 0)),
            pl.BlockSpec((_D, mlp), lambda i: (0, 0)),
            pl.BlockSpec((1, mlp), lambda i: (0, 0)),
            pl.BlockSpec((mlp, c), lambda i: (0, 0)),
            pl.BlockSpec((1, c), lambda i: (0, 0)),
        ],
        out_specs=pl.BlockSpec((_R, c), lambda i: (i, 0)),
        out_shape=jax.ShapeDtypeStruct((_N, c), jnp.float32),
    )(h2, batch2d, m, fat, fbt, fb1, fw2t, fb2)


def kernel(x, edge_index, edge_attr, batch, mask, W1, b1, W2, b2,
           g1, be1, g2, be2, fW1, fb1, fW2, fb2):
    del mask
    pad = _EPAD - _E
    src_r = jnp.concatenate(
        [edge_index[0], jnp.zeros((pad,), jnp.int32)]).reshape(_NW, _CPW, _CHUNK)
    dst_r = jnp.concatenate(
        [edge_index[1], jnp.full((pad,), _N, jnp.int32)]).reshape(_NW, _CPW, _CHUNK)
    pad_e = _EPADE - _E
    dst_re = jnp.concatenate(
        [edge_index[1], jnp.full((pad_e,), _N, jnp.int32)]).reshape(
            _NW, _CPWE, _CHUNK)
    ea_aug = jnp.zeros((_EPADE, _TW), jnp.float32)
    ea_aug = ea_aug.at[:_E, :_DE].set(edge_attr).at[:_E, _DE].set(1.0)
    ea_r = ea_aug.reshape(_NW, _CPWE, _CHUNK, _TW)

    wd1t, ws1t, we1t = W1[:, :_D].T, W1[:, _D:2 * _D].T, W1[:, 2 * _D:].T
    wd2t, ws2t, we2t = W2[:, :_D].T, W2[:, _D:2 * _D].T, W2[:, 2 * _D:].T
    fat, fbt = fW1[:, :_D].T, fW1[:, _D:].T
    fw2t = fW2.T
    batch2d = batch.reshape(_N, 1)

    s1_pair = _make_row_scatter()(x, src_r, dst_r)
    t_pair = _make_ea_scatter()(ea_r, dst_re)
    hpre1, stats1 = _layer_dense(x, s1_pair, t_pair, wd1t, ws1t, we1t,
                                 b1.reshape(1, _D))
    h1 = _bn_apply(hpre1, stats1, g1.reshape(1, _D), be1.reshape(1, _D))

    s2_pair = _make_row_scatter()(h1, src_r, dst_r)
    hpre2, stats2 = _layer_dense(h1, s2_pair, t_pair, wd2t, ws2t, we2t,
                                 b2.reshape(1, _D))
    h2, m = _bn_pool(hpre2, stats2, g2.reshape(1, _D), be2.reshape(1, _D),
                     batch2d)
    return _final(h2, batch2d, m, fat, fbt, fb1.reshape(1, -1), fw2t,
                  fb2.reshape(1, -1))


# speedup vs baseline: 1.2436x; 1.2436x over previous
import functools

import jax
import jax.numpy as jnp
from jax import lax
from jax.experimental import pallas as pl
from jax.experimental.pallas import tpu as pltpu
from jax.experimental.pallas import tpu_sc as plsc

_N = 10000
_E = 320000
_D = 128
_DE = 16
_NB = 64
_EPS = 1e-5
_NC = 2
_NS = 16
_NW = _NC * _NS
_CHUNK = 128
_CPW = 80
_EPW = _CPW * _CHUNK
_EPAD = _NW * _EPW
_HALF = _CPW // 2
_NSH = 10112
_RPT = _NSH // _NS
_ZCH = _NSH // _CHUNK
_TW = 32
_R = 1000
_G = _N // _R

def _mesh():
    return plsc.VectorSubcoreMesh(
        core_axis_name="c", subcore_axis_name="s",
        num_cores=_NC, num_subcores=_NS)


def _zero_buf(buf, width):
    def zrow(i, carry):
        for j in range(width // 16):
            buf[i, pl.ds(j * 16, 16)] = jnp.zeros((16,), jnp.float32)
        return carry
    lax.fori_loop(0, _CHUNK, zrow, 0)


def _zero_shared(s, zbuf, sh):
    def zch(t, carry):
        ch = s + t * _NS

        @pl.when(ch < _ZCH)
        def _():
            pltpu.sync_copy(zbuf, sh.at[pl.ds(ch * _CHUNK, _CHUNK)])
        return carry
    lax.fori_loop(0, (_ZCH + _NS - 1) // _NS, zch, 0)


@functools.lru_cache(maxsize=None)
def _make_row_scatter():
    scratch = [
        pltpu.VMEM((2, _CHUNK), jnp.int32),
        pltpu.VMEM((2, _CHUNK), jnp.int32),
        pltpu.VMEM((_CHUNK, _D), jnp.float32),
        pltpu.VMEM((_CHUNK, _D), jnp.float32),
        pltpu.VMEM_SHARED((_NSH, _D), jnp.float32),
        pltpu.SemaphoreType.DMA,
        pltpu.SemaphoreType.DMA,
    ]

    def body(x_hbm, src_hbm, dst_hbm, s_out,
             src_v, dst_v, rows0, rows1, s_sh, sem0, sem1):
        c = lax.axis_index("c")
        s = lax.axis_index("s")
        wid = s * _NC + c

        _zero_buf(rows0, _D)
        _zero_shared(s, rows0, s_sh)
        plsc.subcore_barrier()

        pltpu.sync_copy(src_hbm.at[wid, 0], src_v.at[0])
        pltpu.sync_copy(dst_hbm.at[wid, 0], dst_v.at[0])
        pltpu.sync_copy(src_hbm.at[wid, 1], src_v.at[1])
        pltpu.sync_copy(dst_hbm.at[wid, 1], dst_v.at[1])
        pltpu.async_copy(x_hbm.at[src_v.at[0]], rows0, sem0)

        def step(t, carry):
            j0 = 2 * t
            j1 = j0 + 1
            pltpu.async_copy(x_hbm.at[src_v.at[1]], rows1, sem1)
            pltpu.make_async_copy(x_hbm.at[src_v.at[0]], rows0, sem0).wait()
            pltpu.sync_copy(rows0, s_sh.at[dst_v.at[0]], add=True)

            @pl.when(t + 1 < _HALF)
            def _():
                pltpu.sync_copy(src_hbm.at[wid, j0 + 2], src_v.at[0])
                pltpu.sync_copy(dst_hbm.at[wid, j0 + 2], dst_v.at[0])
                pltpu.async_copy(x_hbm.at[src_v.at[0]], rows0, sem0)
            pltpu.make_async_copy(x_hbm.at[src_v.at[1]], rows1, sem1).wait()
            pltpu.sync_copy(rows1, s_sh.at[dst_v.at[1]], add=True)

            @pl.when(t + 1 < _HALF)
            def _():
                pltpu.sync_copy(src_hbm.at[wid, j1 + 2], src_v.at[1])
                pltpu.sync_copy(dst_hbm.at[wid, j1 + 2], dst_v.at[1])
            return carry
        lax.fori_loop(0, _HALF, step, 0)
        plsc.subcore_barrier()

        pltpu.sync_copy(s_sh.at[pl.ds(s * _RPT, _RPT)],
                        s_out.at[c, pl.ds(s * _RPT, _RPT)])

    return pl.kernel(
        body, mesh=_mesh(),
        out_type=jax.ShapeDtypeStruct((_NC, _NSH, _D), jnp.float32),
        scratch_types=scratch)


@functools.lru_cache(maxsize=None)
def _make_ea_scatter():
    scratch = [
        pltpu.VMEM((_CPW, _CHUNK), jnp.int32),
        pltpu.VMEM((_CHUNK, _TW), jnp.float32),
        pltpu.VMEM((_CHUNK, _TW), jnp.float32),
        pltpu.VMEM_SHARED((_NSH, _TW), jnp.float32),
        pltpu.SemaphoreType.DMA,
        pltpu.SemaphoreType.DMA,
    ]

    def body(ea_hbm, dst_hbm, t_out, dst_v, ea0, ea1, t_sh, sem0, sem1):
        c = lax.axis_index("c")
        s = lax.axis_index("s")
        wid = s * _NC + c

        _zero_buf(ea0, _TW)
        pltpu.sync_copy(dst_hbm.at[wid], dst_v)
        _zero_shared(s, ea0, t_sh)
        plsc.subcore_barrier()

        pltpu.async_copy(ea_hbm.at[wid, 0], ea0, sem0)

        def step(t, carry):
            j0 = 2 * t
            j1 = j0 + 1
            pltpu.async_copy(ea_hbm.at[wid, j1], ea1, sem1)
            pltpu.make_async_copy(ea_hbm.at[wid, j0], ea0, sem0).wait()
            pltpu.sync_copy(ea0, t_sh.at[dst_v.at[j0]], add=True)

            @pl.when(t + 1 < _HALF)
            def _():
                pltpu.async_copy(ea_hbm.at[wid, j0 + 2], ea0, sem0)
            pltpu.make_async_copy(ea_hbm.at[wid, j1], ea1, sem1).wait()
            pltpu.sync_copy(ea1, t_sh.at[dst_v.at[j1]], add=True)
            return carry
        lax.fori_loop(0, _HALF, step, 0)
        plsc.subcore_barrier()

        pltpu.sync_copy(t_sh.at[pl.ds(s * _RPT, _RPT)],
                        t_out.at[c, pl.ds(s * _RPT, _RPT)])

    return pl.kernel(
        body, mesh=_mesh(),
        out_type=jax.ShapeDtypeStruct((_NC, _NSH, _TW), jnp.float32),
        scratch_types=scratch)


def _dense_body(x_ref, s_ref, t_ref, wd_ref, ws_ref, we_ref, b_ref,
                hpre_ref, stats_ref, sum_ref, sq_ref):
    i = pl.program_id(0)
    x = x_ref[...]
    ssum = s_ref[0] + s_ref[1] + x
    tb = t_ref[0] + t_ref[1]
    deg = tb[:, 16:17] + 1.0
    eap = tb[:, :16] + 1.0
    agg = deg * (jnp.dot(x, wd_ref[...], preferred_element_type=jnp.float32,
                         precision=lax.Precision.HIGHEST) + b_ref[...])
    agg = agg + jnp.dot(ssum, ws_ref[...], preferred_element_type=jnp.float32,
                        precision=lax.Precision.HIGHEST)
    agg = agg + jnp.dot(eap, we_ref[...], preferred_element_type=jnp.float32,
                        precision=lax.Precision.HIGHEST)
    hp = jnp.maximum(agg, 0.0)
    hpre_ref[...] = hp

    @pl.when(i == 0)
    def _():
        sum_ref[...] = jnp.zeros_like(sum_ref)
        sq_ref[...] = jnp.zeros_like(sq_ref)
    sum_ref[...] += jnp.sum(hp, axis=0, keepdims=True)
    sq_ref[...] += jnp.sum(hp * hp, axis=0, keepdims=True)

    @pl.when(i == _G - 1)
    def _():
        stats_ref[...] = jnp.concatenate([sum_ref[...], sq_ref[...]], axis=0)


def _layer_dense(x, s_pair, t_pair, wdt, wst, wet, b):
    return pl.pallas_call(
        _dense_body,
        grid=(_G,),
        in_specs=[
            pl.BlockSpec((_R, _D), lambda i: (i, 0)),
            pl.BlockSpec((2, _R, _D), lambda i: (0, i, 0)),
            pl.BlockSpec((2, _R, _TW), lambda i: (0, i, 0)),
            pl.BlockSpec((_D, _D), lambda i: (0, 0)),
            pl.BlockSpec((_D, _D), lambda i: (0, 0)),
            pl.BlockSpec((_DE, _D), lambda i: (0, 0)),
            pl.BlockSpec((1, _D), lambda i: (0, 0)),
        ],
        out_specs=[
            pl.BlockSpec((_R, _D), lambda i: (i, 0)),
            pl.BlockSpec((2, _D), lambda i: (0, 0)),
        ],
        out_shape=[
            jax.ShapeDtypeStruct((_N, _D), jnp.float32),
            jax.ShapeDtypeStruct((2, _D), jnp.float32),
        ],
        scratch_shapes=[
            pltpu.VMEM((1, _D), jnp.float32),
            pltpu.VMEM((1, _D), jnp.float32),
        ],
    )(x, s_pair, t_pair, wdt, wst, wet, b)


def _bn_scale_shift(stats, g, be):
    mu = stats[0:1] / _N
    var = stats[1:2] / _N - mu * mu
    inv = g * lax.rsqrt(var + _EPS)
    return inv, be - mu * inv


def _bn_body(hp_ref, st_ref, g_ref, be_ref, h_ref):
    inv, shift = _bn_scale_shift(st_ref[...], g_ref[...], be_ref[...])
    h_ref[...] = jnp.maximum(hp_ref[...] * inv + shift, 0.0)


def _bn_apply(hpre, stats, g, be):
    return pl.pallas_call(
        _bn_body,
        grid=(_G,),
        in_specs=[
            pl.BlockSpec((_R, _D), lambda i: (i, 0)),
            pl.BlockSpec((2, _D), lambda i: (0, 0)),
            pl.BlockSpec((1, _D), lambda i: (0, 0)),
            pl.BlockSpec((1, _D), lambda i: (0, 0)),
        ],
        out_specs=pl.BlockSpec((_R, _D), lambda i: (i, 0)),
        out_shape=jax.ShapeDtypeStruct((_N, _D), jnp.float32),
    )(hpre, stats, g, be)


def _onehot(batch_block):
    io = lax.broadcasted_iota(jnp.int32, (_R, _NB), 1)
    return (batch_block == io).astype(jnp.float32)


def _bn_pool_body(hp_ref, st_ref, g_ref, be_ref, bt_ref, h_ref, m_ref, macc):
    i = pl.program_id(0)
    inv, shift = _bn_scale_shift(st_ref[...], g_ref[...], be_ref[...])
    h = jnp.maximum(hp_ref[...] * inv + shift, 0.0)
    h_ref[...] = h
    oh = _onehot(bt_ref[...])
    contrib = lax.dot_general(oh, h, (((0,), (0,)), ((), ())),
                              preferred_element_type=jnp.float32,
                              precision=lax.Precision.HIGHEST)

    @pl.when(i == 0)
    def _():
        macc[...] = jnp.zeros_like(macc)
    macc[...] += contrib

    @pl.when(i == _G - 1)
    def _():
        m_ref[...] = macc[...]


def _bn_pool(hpre, stats, g, be, batch2d):
    return pl.pallas_call(
        _bn_pool_body,
        grid=(_G,),
        in_specs=[
            pl.BlockSpec((_R, _D), lambda i: (i, 0)),
            pl.BlockSpec((2, _D), lambda i: (0, 0)),
            pl.BlockSpec((1, _D), lambda i: (0, 0)),
            pl.BlockSpec((1, _D), lambda i: (0, 0)),
            pl.BlockSpec((_R, 1), lambda i: (i, 0)),
        ],
        out_specs=[
            pl.BlockSpec((_R, _D), lambda i: (i, 0)),
            pl.BlockSpec((_NB, _D), lambda i: (0, 0)),
        ],
        out_shape=[
            jax.ShapeDtypeStruct((_N, _D), jnp.float32),
            jax.ShapeDtypeStruct((_NB, _D), jnp.float32),
        ],
        scratch_shapes=[pltpu.VMEM((_NB, _D), jnp.float32)],
    )(hpre, stats, g, be, batch2d)


def _final_body(h_ref, bt_ref, m_ref, fa_ref, fb_ref, fb1_ref, fw2_ref,
                fb2_ref, out_ref):
    oh = _onehot(bt_ref[...])
    rep = jnp.dot(oh, m_ref[...], preferred_element_type=jnp.float32,
                  precision=lax.Precision.HIGHEST)
    z = jnp.dot(h_ref[...], fa_ref[...], preferred_element_type=jnp.float32,
                precision=lax.Precision.HIGHEST)
    z = z + jnp.dot(rep, fb_ref[...], preferred_element_type=jnp.float32,
                    precision=lax.Precision.HIGHEST)
    z = jnp.maximum(z + fb1_ref[...], 0.0)
    out_ref[...] = jnp.dot(z, fw2_ref[...], preferred_element_type=jnp.float32,
                           precision=lax.Precision.HIGHEST) + fb2_ref[...]


def _final(h2, batch2d, m, fat, fbt, fb1, fw2t, fb2):
    mlp = fat.shape[1]
    c = fw2t.shape[1]
    return pl.pallas_call(
        _final_body,
        grid=(_G,),
        in_specs=[
            pl.BlockSpec((_R, _D), lambda i: (i, 0)),
            pl.BlockSpec((_R, 1), lambda i: (i, 0)),
            pl.BlockSpec((_NB, _D), lambda i: (0, 0)),
            pl.BlockSpec((_D, mlp), lambda i: (0, 0)),
            pl.BlockSpec((_D, mlp), lambda i: (0, 0)),
            pl.BlockSpec((1, mlp), lambda i: (0, 0)),
            pl.BlockSpec((mlp, c), lambda i: (0, 0)),
            pl.BlockSpec((1, c), lambda i: (0, 0)),
        ],
        out_specs=pl.BlockSpec((_R, c), lambda i: (i, 0)),
        out_shape=jax.ShapeDtypeStruct((_N, c), jnp.float32),
    )(h2, batch2d, m, fat, fbt, fb1, fw2t, fb2)


def kernel(x, edge_index, edge_attr, batch, mask, W1, b1, W2, b2,
           g1, be1, g2, be2, fW1, fb1, fW2, fb2):
    del mask
    pad = _EPAD - _E
    src_r = jnp.concatenate(
        [edge_index[0], jnp.zeros((pad,), jnp.int32)]).reshape(_NW, _CPW, _CHUNK)
    dst_r = jnp.concatenate(
        [edge_index[1], jnp.full((pad,), _N, jnp.int32)]).reshape(_NW, _CPW, _CHUNK)
    ea_aug = jnp.zeros((_EPAD, _TW), jnp.float32)
    ea_aug = ea_aug.at[:_E, :_DE].set(edge_attr).at[:_E, _DE].set(1.0)
    ea_r = ea_aug.reshape(_NW, _CPW, _CHUNK, _TW)

    wd1t, ws1t, we1t = W1[:, :_D].T, W1[:, _D:2 * _D].T, W1[:, 2 * _D:].T
    wd2t, ws2t, we2t = W2[:, :_D].T, W2[:, _D:2 * _D].T, W2[:, 2 * _D:].T
    fat, fbt = fW1[:, :_D].T, fW1[:, _D:].T
    fw2t = fW2.T
    batch2d = batch.reshape(_N, 1)

    s1_pair = _make_row_scatter()(x, src_r, dst_r)
    t_pair = _make_ea_scatter()(ea_r, dst_r)
    hpre1, stats1 = _layer_dense(x, s1_pair, t_pair, wd1t, ws1t, we1t,
                                 b1.reshape(1, _D))
    h1 = _bn_apply(hpre1, stats1, g1.reshape(1, _D), be1.reshape(1, _D))

    s2_pair = _make_row_scatter()(h1, src_r, dst_r)
    hpre2, stats2 = _layer_dense(h1, s2_pair, t_pair, wd2t, ws2t, we2t,
                                 b2.reshape(1, _D))
    h2, m = _bn_pool(hpre2, stats2, g2.reshape(1, _D), be2.reshape(1, _D),
                     batch2d)
    return _final(h2, batch2d, m, fat, fbt, fb1.reshape(1, -1), fw2t,
                  fb2.reshape(1, -1))
